# baseline (speedup 1.0000x reference)
import functools

import jax
import jax.numpy as jnp
from jax import lax
from jax.experimental import pallas as pl
from jax.experimental.pallas import tpu as pltpu
from jax.experimental.pallas import tpu_sc as plsc


def kernel(frames):
    B, C, T, H, W = frames.shape
    S = T // 4
    ROWS = B * C * S
    NW = 32
    RPW = ROWS // NW
    D = H * W

    flat = frames.reshape(B * C * T, D)
    mesh = plsc.VectorSubcoreMesh(core_axis_name="c", subcore_axis_name="s")

    @functools.partial(
        pl.kernel,
        out_type=jax.ShapeDtypeStruct((ROWS, D), frames.dtype),
        mesh=mesh,
        scratch_types=[pltpu.SemaphoreType.DMA],
    )
    def pack_slow(src_hbm, out_hbm, sem):
        wid = lax.axis_index("s") * 2 + lax.axis_index("c")
        base = wid * RPW
        copies = []
        for i in range(RPW):
            r = base + i
            bc = r // S
            tp = r % S
            src_row = bc * T + (tp * (T - 1)) // (S - 1)
            copies.append(
                pltpu.make_async_copy(src_hbm.at[src_row], out_hbm.at[r], sem)
            )
        for c in copies:
            c.start()
        for c in copies:
            c.wait()

    slow = pack_slow(flat).reshape(B, C, S, H, W)
    return (slow, frames)


# speedup vs baseline: 4.2626x; 4.2626x over previous
import functools

import jax
import jax.numpy as jnp
from jax import lax
from jax.experimental import pallas as pl
from jax.experimental.pallas import tpu as pltpu
from jax.experimental.pallas import tpu_sc as plsc


def kernel(frames):
    B, C, T, H, W = frames.shape
    S = T // 4
    ROWS = B * C * S
    NW = 32
    RPW = ROWS // NW
    D = H * W

    flat = frames.reshape(B * C * T, D)
    mesh = plsc.VectorSubcoreMesh(core_axis_name="c", subcore_axis_name="s")

    @functools.partial(
        pl.kernel,
        out_type=jax.ShapeDtypeStruct((ROWS, D), frames.dtype),
        mesh=mesh,
        scratch_types=[
            pltpu.VMEM((2, D), frames.dtype),
            pltpu.SemaphoreType.DMA,
            pltpu.SemaphoreType.DMA,
            pltpu.SemaphoreType.DMA,
            pltpu.SemaphoreType.DMA,
        ],
    )
    def pack_slow(src_hbm, out_hbm, buf, si0, si1, so0, so1):
        wid = lax.axis_index("s") * 2 + lax.axis_index("c")
        base = wid * RPW
        sin = (si0, si1)
        sout = (so0, so1)

        def gather(i):
            r = base + i
            bc = r // S
            tp = r % S
            src_row = bc * T + (tp * (T - 1)) // (S - 1)
            return pltpu.make_async_copy(src_hbm.at[src_row], buf.at[i % 2],
                                         sin[i % 2])

        def scatter(i):
            return pltpu.make_async_copy(buf.at[i % 2], out_hbm.at[base + i],
                                         sout[i % 2])

        gather(0).start()
        for i in range(RPW):
            if i + 1 < RPW:
                if i >= 1:
                    scatter(i - 1).wait()
                gather(i + 1).start()
            gather(i).wait()
            scatter(i).start()
        scatter(RPW - 2).wait()
        scatter(RPW - 1).wait()

    slow = pack_slow(flat).reshape(B, C, S, H, W)
    return (slow, frames)
